# baseline (speedup 1.0000x reference)
import functools

import jax
import jax.numpy as jnp
from jax import lax
from jax.experimental import pallas as pl
from jax.experimental.pallas import tpu as pltpu
from jax.experimental.pallas import tpu_sc as plsc

D_MODEL = 64
NUM_ROWS = 1000000
F_DIM = 26
G_B = 8


def _gather_call(idx, table2, n_b):
    info = plsc.get_sparse_core_info()
    nw = info.num_cores * info.num_subcores
    b_per_w = n_b // nw
    chunk = G_B * F_DIM
    n_chunks = b_per_w // G_B
    n_pairs = n_chunks // 2
    mesh = plsc.VectorSubcoreMesh(core_axis_name="c", subcore_axis_name="s")

    @functools.partial(
        pl.kernel,
        mesh=mesh,
        out_type=jax.ShapeDtypeStruct((n_b, F_DIM, D_MODEL), jnp.float32),
        scratch_types=[
            pltpu.VMEM((chunk,), jnp.int32),
            pltpu.VMEM((chunk,), jnp.int32),
            pltpu.VMEM((chunk,), jnp.int32),
            pltpu.VMEM((chunk,), jnp.int32),
            pltpu.VMEM((chunk, 128), jnp.float32),
            pltpu.VMEM((chunk, 128), jnp.float32),
            pltpu.VMEM((chunk, D_MODEL), jnp.float32),
            pltpu.VMEM((chunk, D_MODEL), jnp.float32),
            pltpu.SemaphoreType.DMA,
            pltpu.SemaphoreType.DMA,
            pltpu.SemaphoreType.DMA,
            pltpu.SemaphoreType.DMA,
        ],
        compiler_params=pltpu.CompilerParams(needs_layout_passes=False),
    )
    def k(idx_hbm, tbl_hbm, out_hbm, q0, q1, i0, i1, r0, r1, p0, p1,
          g0, g1, w0, w1):
        qb = (q0, q1)
        icb = (i0, i1)
        rows = (r0, r1)
        packed = (p0, p1)
        gsem = (g0, g1)
        wsem = (w0, w1)
        wid = lax.axis_index("s") * info.num_cores + lax.axis_index("c")
        b_base = wid * b_per_w

        def prep(c, b):
            off = (b_base + c * G_B) * F_DIM
            pltpu.sync_copy(idx_hbm.at[pl.ds(off, chunk)], icb[b])
            for t in range(chunk // 16):
                sl = pl.ds(t * 16, 16)
                qb[b][sl] = icb[b][sl] >> 1

        def gather_start(b):
            pltpu.make_async_copy(tbl_hbm.at[qb[b]], rows[b], gsem[b]).start()

        def gather_wait(b):
            pltpu.make_async_copy(tbl_hbm.at[qb[b]], rows[b], gsem[b]).wait()

        def select(b):
            iota16 = lax.iota(jnp.int32, 16)

            def group_body(g, carry):
                i16 = g * 16 + iota16
                iv = icb[b][pl.ds(g * 16, 16)]
                base = (iv & 1) * 64

                def col_body(col, carry2):
                    v = plsc.load_gather(rows[b], [i16, base + col])
                    plsc.store_scatter(
                        packed[b], [i16, jnp.full((16,), 0, jnp.int32) + col], v
                    )
                    return carry2

                return lax.fori_loop(0, D_MODEL, col_body, carry, unroll=8)

            lax.fori_loop(0, chunk // 16, group_body, 0)

        def write_starts(c, b):
            b0 = b_base + c * G_B
            for g in range(G_B):
                pltpu.make_async_copy(
                    packed[b].at[pl.ds(g * F_DIM, F_DIM)],
                    out_hbm.at[b0 + g],
                    wsem[b],
                ).start()

        def write_waits(c, b):
            b0 = b_base + c * G_B
            for g in range(G_B):
                pltpu.make_async_copy(
                    packed[b].at[pl.ds(g * F_DIM, F_DIM)],
                    out_hbm.at[b0 + g],
                    wsem[b],
                ).wait()

        for b in range(2):
            prep(b, b)
            gather_start(b)

        def pair_body(rr, carry):
            for b in range(2):
                c = rr * 2 + b

                gather_wait(b)

                @pl.when(rr > 0)
                def _():
                    write_waits(c, b)

                select(b)
                write_starts(c, b)

                @pl.when(rr < n_pairs - 1)
                def _():
                    prep(c + 2, b)
                    gather_start(b)
            return carry

        lax.fori_loop(0, n_pairs, pair_body, 0)

        for b in range(2):
            write_waits(0, b)

    return k(idx, table2)


def kernel(x, table):
    n_b, f, _ = x.shape
    idx = x.reshape(n_b * f)
    table2 = table[:NUM_ROWS].reshape(NUM_ROWS // 2, 128)
    out = _gather_call(idx, table2, n_b)
    return out.reshape(n_b, f, 1, D_MODEL)


# speedup vs baseline: 1.9745x; 1.9745x over previous
import functools

import jax
import jax.numpy as jnp
from jax import lax
from jax.experimental import pallas as pl
from jax.experimental.pallas import tpu as pltpu
from jax.experimental.pallas import tpu_sc as plsc

D_MODEL = 64
NUM_ROWS = 1000000
F_DIM = 26
G_B = 8


def _gather_call(idx, table_p, n_b):
    info = plsc.get_sparse_core_info()
    nw = info.num_cores * info.num_subcores
    b_per_w = n_b // nw
    chunk = G_B * F_DIM
    n_chunks = b_per_w // G_B
    n_pairs = n_chunks // 2
    mesh = plsc.VectorSubcoreMesh(core_axis_name="c", subcore_axis_name="s")

    @functools.partial(
        pl.kernel,
        mesh=mesh,
        out_type=jax.ShapeDtypeStruct((n_b, F_DIM, D_MODEL), jnp.float32),
        scratch_types=[
            pltpu.VMEM((chunk,), jnp.int32),
            pltpu.VMEM((chunk,), jnp.int32),
            pltpu.VMEM((chunk, 128), jnp.float32),
            pltpu.VMEM((chunk, 128), jnp.float32),
            pltpu.VMEM((chunk, D_MODEL), jnp.float32),
            pltpu.VMEM((chunk, D_MODEL), jnp.float32),
            pltpu.SemaphoreType.DMA,
            pltpu.SemaphoreType.DMA,
            pltpu.SemaphoreType.DMA,
            pltpu.SemaphoreType.DMA,
        ],
    )
    def k(idx_hbm, tbl_hbm, out_hbm, q0, q1, r0, r1, p0, p1, g0, g1, w0, w1):
        qb = (q0, q1)
        rows = (r0, r1)
        packed = (p0, p1)
        gsem = (g0, g1)
        wsem = (w0, w1)
        wid = lax.axis_index("s") * info.num_cores + lax.axis_index("c")
        b_base = wid * b_per_w

        def prep(c, b):
            off = (b_base + c * G_B) * F_DIM
            pltpu.sync_copy(idx_hbm.at[pl.ds(off, chunk)], qb[b])

        def gather_start(b):
            pltpu.make_async_copy(tbl_hbm.at[qb[b]], rows[b], gsem[b]).start()

        def gather_wait(b):
            pltpu.make_async_copy(tbl_hbm.at[qb[b]], rows[b], gsem[b]).wait()

        def select(b):
            def group_body(i, carry):
                for j in range(4):
                    sl = pl.ds(j * 16, 16)
                    packed[b][i, sl] = rows[b][i, sl]
                return carry
            lax.fori_loop(0, chunk, group_body, 0, unroll=8)

        def write_starts(c, b):
            b0 = b_base + c * G_B
            for g in range(G_B):
                pltpu.make_async_copy(
                    packed[b].at[pl.ds(g * F_DIM, F_DIM)],
                    out_hbm.at[b0 + g],
                    wsem[b],
                ).start()

        def write_waits(c, b):
            b0 = b_base + c * G_B
            for g in range(G_B):
                pltpu.make_async_copy(
                    packed[b].at[pl.ds(g * F_DIM, F_DIM)],
                    out_hbm.at[b0 + g],
                    wsem[b],
                ).wait()

        for b in range(2):
            prep(b, b)
            gather_start(b)

        def pair_body(rr, carry):
            for b in range(2):
                c = rr * 2 + b

                gather_wait(b)

                @pl.when(rr > 0)
                def _():
                    write_waits(c, b)

                select(b)
                write_starts(c, b)

                @pl.when(rr < n_pairs - 1)
                def _():
                    prep(c + 2, b)
                    gather_start(b)
            return carry

        lax.fori_loop(0, n_pairs, pair_body, 0)

        for b in range(2):
            write_waits(0, b)

    return k(idx, table_p)


def kernel(x, table):
    n_b, f, _ = x.shape
    idx = x.reshape(n_b * f)
    table_p = jnp.pad(table[:NUM_ROWS], ((0, 0), (0, 128 - D_MODEL)))
    out = _gather_call(idx, table_p, n_b)
    return out.reshape(n_b, f, 1, D_MODEL)
